# baseline (speedup 1.0000x reference)
import functools

import jax
import jax.numpy as jnp
from jax import lax
from jax.experimental import pallas as pl
from jax.experimental.pallas import tpu as pltpu
from jax.experimental.pallas import tpu_sc as plsc

NUM_CLASSES = 4
MARGIN = 1.0

B = 2
C = 32
N = 64 * 64 * 64
K = NUM_CLASSES
L = 16
NC = 2
NS = 16
NW = NC * NS
CHUNK = N // NW
NVEC = CHUNK // L
SLOT = K * L
NSLOT = C + 1
PB = NSLOT * SLOT
PTOT = B * PB


def _sc_body(feat_hbm, tgt_hbm, out_hbm, idx_v, fbuf, acc_v, sem):
    wid = lax.axis_index("s") * NC + lax.axis_index("c")
    base = wid * CHUNK

    pltpu.sync_copy(tgt_hbm.at[0, pl.ds(base, CHUNK)], idx_v.at[pl.ds(0, CHUNK)])
    pltpu.sync_copy(tgt_hbm.at[1, pl.ds(base, CHUNK)], idx_v.at[pl.ds(CHUNK, CHUNK)])

    zeros = jnp.zeros((L,), jnp.float32)

    def zbody(i, carry):
        acc_v[pl.ds(i * L, L)] = zeros
        return carry

    lax.fori_loop(0, PTOT // L, zbody, 0)

    lane = lax.iota(jnp.int32, L)
    ones = jnp.ones((L,), jnp.float32)

    def ibody(i, carry):
        t = idx_v[pl.ds(i * L, L)]
        bb = i // NVEC
        idx = t * L + lane + bb * PB
        idx_v[pl.ds(i * L, L)] = idx
        plsc.addupdate_scatter(acc_v, [idx + C * SLOT], ones)
        return carry

    lax.fori_loop(0, B * NVEC, ibody, 0)

    def fbody(j, carry):
        bb = j // C
        cc = j % C
        pltpu.sync_copy(feat_hbm.at[bb, cc, pl.ds(base, CHUNK)], fbuf)
        coff = cc * SLOT

        def vbody(i, c2):
            f = fbuf[pl.ds(i * L, L)]
            iv = idx_v[pl.ds(bb * CHUNK + i * L, L)]
            plsc.addupdate_scatter(acc_v, [iv + coff], f)
            return c2

        lax.fori_loop(0, NVEC, vbody, 0)
        return carry

    lax.fori_loop(0, B * C, fbody, 0)

    pltpu.sync_copy(acc_v, out_hbm.at[wid])


_sc_call = functools.partial(
    pl.kernel,
    mesh=plsc.VectorSubcoreMesh(core_axis_name="c", subcore_axis_name="s"),
    out_type=jax.ShapeDtypeStruct((NW, PTOT), jnp.float32),
    scratch_types=[
        pltpu.VMEM((B * CHUNK,), jnp.int32),
        pltpu.VMEM((CHUNK,), jnp.float32),
        pltpu.VMEM((PTOT,), jnp.float32),
        pltpu.SemaphoreType.DMA,
    ],
    compiler_params=pltpu.CompilerParams(needs_layout_passes=False),
)(_sc_body)


def _fin_body(p_ref, loss_ref):
    p = p_ref[...]
    s = jnp.sum(p, axis=(0, 4))
    sums = s[:, :C, :]
    counts = s[:, C, :]
    centers = sums / jnp.maximum(counts, 1.0)[:, None, :]
    valid = counts > 0
    total = jnp.float32(0.0)
    pairs = jnp.float32(0.0)
    for i in range(NUM_CLASSES):
        for j in range(i + 1, NUM_CLASSES):
            diff = centers[:, :, i] - centers[:, :, j]
            dist = jnp.sqrt(jnp.sum(diff * diff, axis=1))
            hinge = jnp.maximum(MARGIN - dist, 0.0)
            m = jnp.logical_and(valid[:, i], valid[:, j]).astype(jnp.float32)
            total = total + jnp.sum(hinge * m)
            pairs = pairs + jnp.sum(m)
    val = jnp.where(pairs > 0, total / jnp.maximum(pairs, 1.0), 0.0)
    loss_ref[...] = val.reshape(1, 1)


def _finalize(q):
    return pl.pallas_call(
        _fin_body,
        out_shape=jax.ShapeDtypeStruct((1, 1), jnp.float32),
    )(q)


def kernel(features, predictions, targets):
    del predictions
    feats = features.reshape(B, C, N)
    tgt = targets.reshape(B, N)
    partial = _sc_call(feats, tgt)
    q = partial.reshape(NW, B, NSLOT, K, L)
    loss = _finalize(q)
    return loss[0, 0]


# speedup vs baseline: 1.5575x; 1.5575x over previous
import functools

import jax
import jax.numpy as jnp
from jax import lax
from jax.experimental import pallas as pl
from jax.experimental.pallas import tpu as pltpu
from jax.experimental.pallas import tpu_sc as plsc

NUM_CLASSES = 4
MARGIN = 1.0

B = 2
C = 32
N = 64 * 64 * 64
K = NUM_CLASSES
L = 16
NC = 2
NS = 16
NW = NC * NS
CHUNK = N // NW
NVEC = CHUNK // L
SLOT = K * L
NSLOT = C + 1
PB = NSLOT * SLOT
PTOT = B * PB


GC = 4
NG = B * C // GC
GPB = C // GC
UN = 2


def _sc_body(feat_hbm, tgt_hbm, out_hbm, idx_v, fbuf, acc_v, sem0, sem1):
    wid = lax.axis_index("s") * NC + lax.axis_index("c")
    base = wid * CHUNK
    sems = (sem0, sem1)

    tcp0 = pltpu.async_copy(
        tgt_hbm.at[0, pl.ds(base, CHUNK)], idx_v.at[pl.ds(0, CHUNK)], sem0
    )
    tcp1 = pltpu.async_copy(
        tgt_hbm.at[1, pl.ds(base, CHUNK)], idx_v.at[pl.ds(CHUNK, CHUNK)], sem1
    )

    zeros = jnp.zeros((L,), jnp.float32)

    def zbody(i, carry):
        for u in range(4):
            acc_v[pl.ds((i * 4 + u) * L, L)] = zeros
        return carry

    lax.fori_loop(0, PTOT // (4 * L), zbody, 0)
    tcp0.wait()
    tcp1.wait()

    lane = lax.iota(jnp.int32, L)
    ones = jnp.ones((L,), jnp.float32)

    for bb in range(B):

        def ibody(i, carry):
            for u in range(UN):
                sl = pl.ds(bb * CHUNK + (i * UN + u) * L, L)
                idx = idx_v[sl] * L + lane + (bb * PB)
                idx_v[sl] = idx
                plsc.addupdate_scatter(acc_v, [idx + C * SLOT], ones)
            return carry

        lax.fori_loop(0, NVEC // UN, ibody, 0)

    def start(g):
        bb = g // GPB
        c0 = (g % GPB) * GC
        return pltpu.async_copy(
            feat_hbm.at[bb, pl.ds(c0, GC), pl.ds(base, CHUNK)],
            fbuf.at[g % 2],
            sems[g % 2],
        )

    cur = start(0)
    for g in range(NG):
        nxt = start(g + 1) if g + 1 < NG else None
        cur.wait()
        bb = g // GPB
        c0 = (g % GPB) * GC
        buf = g % 2
        boff = bb * CHUNK

        def vbody(i, carry, buf=buf, boff=boff, c0=c0):
            for u in range(UN):
                voff = (i * UN + u) * L
                iv = idx_v[pl.ds(boff + voff, L)]
                for cs in range(GC):
                    f = fbuf[buf, cs, pl.ds(voff, L)]
                    plsc.addupdate_scatter(acc_v, [iv + (c0 + cs) * SLOT], f)
            return carry

        lax.fori_loop(0, NVEC // UN, vbody, 0)
        cur = nxt

    pltpu.sync_copy(acc_v, out_hbm.at[wid])


_sc_call = functools.partial(
    pl.kernel,
    mesh=plsc.VectorSubcoreMesh(core_axis_name="c", subcore_axis_name="s"),
    out_type=jax.ShapeDtypeStruct((NW, PTOT), jnp.float32),
    scratch_types=[
        pltpu.VMEM((B * CHUNK,), jnp.int32),
        pltpu.VMEM((2, GC, CHUNK), jnp.float32),
        pltpu.VMEM((PTOT,), jnp.float32),
        pltpu.SemaphoreType.DMA,
        pltpu.SemaphoreType.DMA,
    ],
    compiler_params=pltpu.CompilerParams(needs_layout_passes=False),
)(_sc_body)


def _fin_body(p_ref, loss_ref):
    p = p_ref[...]
    s = jnp.sum(p, axis=(0, 4))
    sums = s[:, :C, :]
    counts = s[:, C, :]
    centers = sums / jnp.maximum(counts, 1.0)[:, None, :]
    valid = counts > 0
    total = jnp.float32(0.0)
    pairs = jnp.float32(0.0)
    for i in range(NUM_CLASSES):
        for j in range(i + 1, NUM_CLASSES):
            diff = centers[:, :, i] - centers[:, :, j]
            dist = jnp.sqrt(jnp.sum(diff * diff, axis=1))
            hinge = jnp.maximum(MARGIN - dist, 0.0)
            m = jnp.logical_and(valid[:, i], valid[:, j]).astype(jnp.float32)
            total = total + jnp.sum(hinge * m)
            pairs = pairs + jnp.sum(m)
    val = jnp.where(pairs > 0, total / jnp.maximum(pairs, 1.0), 0.0)
    loss_ref[...] = val.reshape(1, 1)


def _finalize(q):
    return pl.pallas_call(
        _fin_body,
        out_shape=jax.ShapeDtypeStruct((1, 1), jnp.float32),
    )(q)


def kernel(features, predictions, targets):
    del predictions
    feats = features.reshape(B, C, N)
    tgt = targets.reshape(B, N)
    partial = _sc_call(feats, tgt)
    q = partial.reshape(NW, B, NSLOT, K, L)
    loss = _finalize(q)
    return loss[0, 0]


# speedup vs baseline: 2.1260x; 1.3649x over previous
import functools

import jax
import jax.numpy as jnp
from jax import lax
from jax.experimental import pallas as pl
from jax.experimental.pallas import tpu as pltpu
from jax.experimental.pallas import tpu_sc as plsc

NUM_CLASSES = 4
MARGIN = 1.0

B = 2
C = 32
N = 64 * 64 * 64
K = NUM_CLASSES
L = 16
NC = 2
NS = 16
NW = NC * NS
CHUNK = N // NW
NVEC = CHUNK // L
SLOT = K * L
NSLOT = C + 1
PB = NSLOT * SLOT
PTOT = B * PB


GC = 4
NG = B * C // GC
GPB = C // GC
R = 4


def _sc_body(feat_hbm, tgt_hbm, out_hbm, idx_v, fbuf, acc_v, sem0, sem1):
    wid = lax.axis_index("s") * NC + lax.axis_index("c")
    base = wid * CHUNK
    sems = (sem0, sem1)

    tcp0 = pltpu.async_copy(
        tgt_hbm.at[0, pl.ds(base, CHUNK)], idx_v.at[pl.ds(0, CHUNK)], sem0
    )
    tcp1 = pltpu.async_copy(
        tgt_hbm.at[1, pl.ds(base, CHUNK)], idx_v.at[pl.ds(CHUNK, CHUNK)], sem1
    )

    zeros = jnp.zeros((L,), jnp.float32)

    @plsc.parallel_loop(0, R * PTOT // L, unroll=4)
    def _zero(i):
        acc_v[pl.ds(i * L, L)] = zeros

    tcp0.wait()
    tcp1.wait()

    lane = lax.iota(jnp.int32, L)
    ones = jnp.ones((L,), jnp.float32)

    for bb in range(B):

        @plsc.parallel_loop(0, NVEC, unroll=4)
        def _prep(i, bb=bb):
            sl = pl.ds(bb * CHUNK + i * L, L)
            idx = idx_v[sl] * L + lane + (bb * PB)
            idx_v[sl] = idx
            roff = (i & (R - 1)) * PTOT
            plsc.addupdate_scatter(acc_v, [idx + (roff + C * SLOT)], ones)

    def start(g):
        bb = g // GPB
        c0 = (g % GPB) * GC
        return pltpu.async_copy(
            feat_hbm.at[bb, pl.ds(c0, GC), pl.ds(base, CHUNK)],
            fbuf.at[g % 2],
            sems[g % 2],
        )

    cur = start(0)
    for g in range(NG):
        nxt = start(g + 1) if g + 1 < NG else None
        cur.wait()
        bb = g // GPB
        c0 = (g % GPB) * GC
        buf = g % 2
        boff = bb * CHUNK

        @plsc.parallel_loop(0, NVEC, unroll=4)
        def _scatter(i, buf=buf, boff=boff, c0=c0):
            voff = i * L
            iv = idx_v[pl.ds(boff + voff, L)]
            ivr = iv + (i & (R - 1)) * PTOT
            for cs in range(GC):
                f = fbuf[buf, cs, pl.ds(voff, L)]
                plsc.addupdate_scatter(acc_v, [ivr + (c0 + cs) * SLOT], f)

        cur = nxt

    pltpu.sync_copy(acc_v, out_hbm.at[wid])


_sc_call = functools.partial(
    pl.kernel,
    mesh=plsc.VectorSubcoreMesh(core_axis_name="c", subcore_axis_name="s"),
    out_type=jax.ShapeDtypeStruct((NW, R * PTOT), jnp.float32),
    scratch_types=[
        pltpu.VMEM((B * CHUNK,), jnp.int32),
        pltpu.VMEM((2, GC, CHUNK), jnp.float32),
        pltpu.VMEM((R * PTOT,), jnp.float32),
        pltpu.SemaphoreType.DMA,
        pltpu.SemaphoreType.DMA,
    ],
    compiler_params=pltpu.CompilerParams(needs_layout_passes=False),
)(_sc_body)


def _fin_body(p_ref, loss_ref):
    p = p_ref[...]
    s = jnp.sum(p, axis=(0, 4))
    sums = s[:, :C, :]
    counts = s[:, C, :]
    centers = sums / jnp.maximum(counts, 1.0)[:, None, :]
    valid = counts > 0
    total = jnp.float32(0.0)
    pairs = jnp.float32(0.0)
    for i in range(NUM_CLASSES):
        for j in range(i + 1, NUM_CLASSES):
            diff = centers[:, :, i] - centers[:, :, j]
            dist = jnp.sqrt(jnp.sum(diff * diff, axis=1))
            hinge = jnp.maximum(MARGIN - dist, 0.0)
            m = jnp.logical_and(valid[:, i], valid[:, j]).astype(jnp.float32)
            total = total + jnp.sum(hinge * m)
            pairs = pairs + jnp.sum(m)
    val = jnp.where(pairs > 0, total / jnp.maximum(pairs, 1.0), 0.0)
    loss_ref[...] = val.reshape(1, 1)


def _finalize(q):
    return pl.pallas_call(
        _fin_body,
        out_shape=jax.ShapeDtypeStruct((1, 1), jnp.float32),
    )(q)


def kernel(features, predictions, targets):
    del predictions
    feats = features.reshape(B, C, N)
    tgt = targets.reshape(B, N)
    partial = _sc_call(feats, tgt)
    q = partial.reshape(NW * R, B, NSLOT, K, L)
    loss = _finalize(q)
    return loss[0, 0]
